# Initial kernel scaffold; baseline (speedup 1.0000x reference)
import jax
import jax.numpy as jnp
from jax.experimental import pallas as pl


def kernel(embedding, x):
    raise NotImplementedError("write your pallas kernel here")


# speedup vs baseline: 3.5496x; 3.5496x over previous
import functools

import jax
import jax.numpy as jnp
from jax import lax
from jax.experimental import pallas as pl
from jax.experimental.pallas import tpu as pltpu
from jax.experimental.pallas import tpu_sc as plsc

_CHUNK = 128


@functools.lru_cache(maxsize=None)
def _build(num_idx, dim):
    info = plsc.get_sparse_core_info()
    nc, ns = info.num_cores, info.num_subcores
    nw = nc * ns
    chunks_total = num_idx // _CHUNK
    chunks_per_w = chunks_total // nw
    per_w = chunks_per_w * _CHUNK

    mesh = plsc.VectorSubcoreMesh(core_axis_name="c", subcore_axis_name="s")

    @functools.partial(
        pl.kernel,
        mesh=mesh,
        compiler_params=pltpu.CompilerParams(use_tc_tiling_on_sc=False),
        out_type=jax.ShapeDtypeStruct((num_idx, dim), jnp.float32),
        scratch_types=[
            pltpu.VMEM((chunks_per_w, _CHUNK), jnp.int32),
            pltpu.VMEM((_CHUNK, dim), jnp.float32),
            pltpu.SemaphoreType.DMA,
        ],
    )
    def k(table_hbm, idx_hbm, out_hbm, idx_v, buf, gsem):
        wid = lax.axis_index("s") * nc + lax.axis_index("c")
        row0 = wid * chunks_per_w
        pltpu.sync_copy(idx_hbm.at[pl.ds(row0, chunks_per_w)], idx_v)
        base = wid * per_w

        def body(j, carry):
            pltpu.async_copy(table_hbm.at[idx_v.at[j]], buf, gsem).wait()
            pltpu.sync_copy(
                buf, out_hbm.at[pl.ds(base + j * _CHUNK, _CHUNK)]
            )
            return carry

        lax.fori_loop(0, chunks_per_w, body, 0)

    return k


def kernel(embedding, x):
    orig_shape = x.shape
    xf = x.reshape(-1)
    n = xf.shape[0]
    idx2d = xf.reshape(n // _CHUNK, _CHUNK)
    out = _build(n, embedding.shape[1])(embedding, idx2d)
    return out.reshape(*orig_shape, embedding.shape[1])


# speedup vs baseline: 4.2564x; 1.1991x over previous
import functools

import jax
import jax.numpy as jnp
from jax import lax
from jax.experimental import pallas as pl
from jax.experimental.pallas import tpu as pltpu
from jax.experimental.pallas import tpu_sc as plsc

_CHUNK = 128
_NBUF = 4


@functools.lru_cache(maxsize=None)
def _build(num_idx, dim):
    info = plsc.get_sparse_core_info()
    nc, ns = info.num_cores, info.num_subcores
    nw = nc * ns
    chunks_total = num_idx // _CHUNK
    n = chunks_total // nw
    per_w = n * _CHUNK

    mesh = plsc.VectorSubcoreMesh(core_axis_name="c", subcore_axis_name="s")

    @functools.partial(
        pl.kernel,
        mesh=mesh,
        compiler_params=pltpu.CompilerParams(use_tc_tiling_on_sc=False),
        out_type=jax.ShapeDtypeStruct((num_idx, dim), jnp.float32),
        scratch_types=[
            pltpu.VMEM((n, _CHUNK), jnp.int32),
        ]
        + [pltpu.VMEM((_CHUNK, dim), jnp.float32) for _ in range(_NBUF)]
        + [pltpu.SemaphoreType.DMA for _ in range(2 * _NBUF)],
    )
    def k(table_hbm, idx_hbm, out_hbm, idx_v, *scr):
        bufs = scr[:_NBUF]
        gsems = scr[_NBUF : 2 * _NBUF]
        ssems = scr[2 * _NBUF :]

        wid = lax.axis_index("s") * nc + lax.axis_index("c")
        row0 = wid * n
        pltpu.sync_copy(idx_hbm.at[pl.ds(row0, n)], idx_v)
        base = wid * per_w

        def g_copy(j, b):
            return pltpu.make_async_copy(
                table_hbm.at[idx_v.at[j]], bufs[b], gsems[b]
            )

        def s_copy(j, b):
            return pltpu.make_async_copy(
                bufs[b], out_hbm.at[pl.ds(base + j * _CHUNK, _CHUNK)], ssems[b]
            )

        def step(j, b):
            g_copy(j, b).wait()
            s_copy(j, b).start()
            pb = (b - 1) % _NBUF
            pj = j - 1
            s_copy(pj, pb).wait()
            g_copy(pj + _NBUF, pb).start()

        for b in range(_NBUF):
            g_copy(b, b).start()
        g_copy(0, 0).wait()
        s_copy(0, 0).start()
        for b in range(1, _NBUF):
            step(b, b)

        def body(o, carry):
            for b in range(_NBUF):
                step(o * _NBUF + b, b)
            return carry

        lax.fori_loop(1, n // _NBUF - 1, body, 0)

        o = n // _NBUF - 1
        step(o * _NBUF, 0)
        for b in range(1, _NBUF):
            j = o * _NBUF + b
            g_copy(j, b).wait()
            s_copy(j, b).start()

        for b in range(_NBUF):
            s_copy(n - _NBUF + b, b).wait()

    return k


def kernel(embedding, x):
    orig_shape = x.shape
    xf = x.reshape(-1)
    num = xf.shape[0]
    idx2d = xf.reshape(num // _CHUNK, _CHUNK)
    out = _build(num, embedding.shape[1])(embedding, idx2d)
    return out.reshape(*orig_shape, embedding.shape[1])
